# baseline (speedup 1.0000x reference)
import jax
import jax.numpy as jnp
from jax import lax
from jax.experimental import pallas as pl
from jax.experimental.pallas import tpu as pltpu

_NEG_RATIO = 3.0
_ALPHA = 1.0
_BETA = 10.0
_EPS = 1e-6
_B = 8
_H = 512
_W = 512


def _safe_log(x):
    return jnp.maximum(jnp.log(jnp.clip(x, 1e-44, 1.0)), -100.0)


def _dbloss_kernel(prob_ref, thr_ref, bin_ref, pgt_ref, sup_ref, tgt_ref,
                   txt_ref, out_ref, bits_ref, acc_ref):
    b = pl.program_id(0)

    @pl.when(b == 0)
    def _init():
        for i in range(6):
            acc_ref[i] = 0.0

    p = prob_ref[0, 0]
    g = pgt_ref[0, 0]
    m = sup_ref[0, 0]
    pos = g * m
    neg = (1.0 - g) * m
    bce = -(g * _safe_log(p) + (1.0 - g) * _safe_log(1.0 - p))
    neg_loss = jnp.abs(bce * neg)
    acc_ref[0] += jnp.sum(bce * pos)
    acc_ref[1] += jnp.sum(pos)
    acc_ref[2] += jnp.sum(neg)
    txt = txt_ref[0, 0]
    acc_ref[3] += jnp.sum(jnp.abs(thr_ref[0, 0] - tgt_ref[0, 0]) * txt)
    acc_ref[4] += jnp.sum(txt)
    pm = bin_ref[0, 0] * m
    gm = g * m
    da = jnp.sum(pm * gm)
    db = jnp.sum(pm * pm) + _EPS
    dc = jnp.sum(gm * gm) + _EPS
    acc_ref[5] += 2.0 * da / (db + dc)

    bits_ref[b] = lax.bitcast_convert_type(neg_loss, jnp.int32)

    @pl.when(b == _B - 1)
    def _finish():
        pos_count = acc_ref[1]
        neg_count = jnp.minimum(acc_ref[2], pos_count * _NEG_RATIO)
        kf = neg_count
        ki = kf.astype(jnp.int32)

        def body(i, prefix):
            cand = prefix | lax.shift_left(jnp.int32(1), 30 - i)
            parts = [jnp.count_nonzero(bits_ref[c][h * 64:(h + 1) * 64] >= cand)
                     for c in range(_B) for h in range(8)]
            cnt = sum(parts)
            return jnp.where(cnt >= ki, cand, prefix)

        t_bits = lax.fori_loop(0, 31, body, jnp.int32(0))
        cnt_parts = []
        sum_parts = []
        for c in range(_B):
            for h in range(2):
                bc = bits_ref[c][h * 256:(h + 1) * 256]
                gm = (bc > t_bits).astype(jnp.float32)
                cnt_parts.append(jnp.sum(gm))
                sum_parts.append(
                    jnp.sum(lax.bitcast_convert_type(bc, jnp.float32) * gm))
        cnt_gt = sum(cnt_parts)
        sum_gt = sum(sum_parts)
        t_val = lax.bitcast_convert_type(t_bits, jnp.float32)
        neg_top = sum_gt + (kf - cnt_gt) * t_val
        neg_top = jnp.where(kf > 0.0, neg_top, 0.0)
        prob_loss = (acc_ref[0] + neg_top) / (pos_count + neg_count + _EPS)
        thr_loss = acc_ref[3] / acc_ref[4]
        bin_loss = 1.0 - acc_ref[5] / _B
        out_ref[0, 0] = prob_loss + _BETA * thr_loss + _ALPHA * bin_loss


@jax.jit
def kernel(preds, gts):
    pred_spec = lambda c: pl.BlockSpec((1, 1, _H, _W), lambda b, c=c: (b, c, 0, 0))
    gt_spec = lambda c: pl.BlockSpec((1, 1, _H, _W), lambda b, c=c: (c, b, 0, 0))
    out = pl.pallas_call(
        _dbloss_kernel,
        grid=(_B,),
        in_specs=[pred_spec(0), pred_spec(1), pred_spec(2),
                  gt_spec(0), gt_spec(1), gt_spec(2), gt_spec(3)],
        out_specs=pl.BlockSpec(memory_space=pltpu.SMEM),
        out_shape=jax.ShapeDtypeStruct((1, 1), jnp.float32),
        scratch_shapes=[
            pltpu.VMEM((_B, _H, _W), jnp.int32),
            pltpu.SMEM((8,), jnp.float32),
        ],
    )(preds, preds, preds, gts, gts, gts, gts)
    return out[0, 0]


# speedup vs baseline: 1.0735x; 1.0735x over previous
import jax
import jax.numpy as jnp
from jax import lax
from jax.experimental import pallas as pl
from jax.experimental.pallas import tpu as pltpu

_NEG_RATIO = 3.0
_ALPHA = 1.0
_BETA = 10.0
_EPS = 1e-6
_B = 8
_H = 512
_W = 512


def _safe_log(x):
    return jnp.maximum(jnp.log(jnp.clip(x, 1e-44, 1.0)), -100.0)


def _dbloss_kernel(prob_ref, thr_ref, bin_ref, pgt_ref, sup_ref, tgt_ref,
                   txt_ref, out_ref, bits_ref, acc_ref):
    b = pl.program_id(0)

    @pl.when(b == 0)
    def _init():
        for i in range(6):
            acc_ref[i] = 0.0

    p = prob_ref[0, 0]
    g = pgt_ref[0, 0]
    m = sup_ref[0, 0]
    pos = g * m
    neg = (1.0 - g) * m
    bce = -(g * _safe_log(p) + (1.0 - g) * _safe_log(1.0 - p))
    neg_loss = jnp.abs(bce * neg)
    acc_ref[0] += jnp.sum(bce * pos)
    acc_ref[1] += jnp.sum(pos)
    acc_ref[2] += jnp.sum(neg)
    txt = txt_ref[0, 0]
    acc_ref[3] += jnp.sum(jnp.abs(thr_ref[0, 0] - tgt_ref[0, 0]) * txt)
    acc_ref[4] += jnp.sum(txt)
    pm = bin_ref[0, 0] * m
    gm = g * m
    da = jnp.sum(pm * gm)
    db = jnp.sum(pm * pm) + _EPS
    dc = jnp.sum(gm * gm) + _EPS
    acc_ref[5] += 2.0 * da / (db + dc)

    bits_ref[b] = lax.bitcast_convert_type(neg_loss, jnp.int32)

    @pl.when(b == _B - 1)
    def _finish():
        pos_count = acc_ref[1]
        neg_count = jnp.minimum(acc_ref[2], pos_count * _NEG_RATIO)
        kf = neg_count
        ki = kf.astype(jnp.int32)

        def body(i, prefix):
            cand = prefix | lax.shift_left(jnp.int32(1), 30 - i)
            parts = [jnp.count_nonzero(bits_ref[c][h * 128:(h + 1) * 128] >= cand)
                     for c in range(_B) for h in range(4)]
            cnt = sum(parts)
            return jnp.where(cnt >= ki, cand, prefix)

        t_bits = lax.fori_loop(0, 31, body, jnp.int32(0))
        cnt_parts = []
        sum_parts = []
        for c in range(_B):
            bc = bits_ref[c]
            gm = (bc > t_bits).astype(jnp.float32)
            cnt_parts.append(jnp.sum(gm))
            sum_parts.append(jnp.sum(lax.bitcast_convert_type(bc, jnp.float32) * gm))
        cnt_gt = sum(cnt_parts)
        sum_gt = sum(sum_parts)
        t_val = lax.bitcast_convert_type(t_bits, jnp.float32)
        neg_top = sum_gt + (kf - cnt_gt) * t_val
        neg_top = jnp.where(kf > 0.0, neg_top, 0.0)
        prob_loss = (acc_ref[0] + neg_top) / (pos_count + neg_count + _EPS)
        thr_loss = acc_ref[3] / acc_ref[4]
        bin_loss = 1.0 - acc_ref[5] / _B
        out_ref[0, 0] = prob_loss + _BETA * thr_loss + _ALPHA * bin_loss


@jax.jit
def kernel(preds, gts):
    pred_spec = lambda c: pl.BlockSpec((1, 1, _H, _W), lambda b, c=c: (b, c, 0, 0))
    gt_spec = lambda c: pl.BlockSpec((1, 1, _H, _W), lambda b, c=c: (c, b, 0, 0))
    out = pl.pallas_call(
        _dbloss_kernel,
        grid=(_B,),
        in_specs=[pred_spec(0), pred_spec(1), pred_spec(2),
                  gt_spec(0), gt_spec(1), gt_spec(2), gt_spec(3)],
        out_specs=pl.BlockSpec(memory_space=pltpu.SMEM),
        out_shape=jax.ShapeDtypeStruct((1, 1), jnp.float32),
        scratch_shapes=[
            pltpu.VMEM((_B, _H, _W), jnp.int32),
            pltpu.SMEM((8,), jnp.float32),
        ],
    )(preds, preds, preds, gts, gts, gts, gts)
    return out[0, 0]
